# baseline (speedup 1.0000x reference)
import functools

import jax
import jax.numpy as jnp
from jax import lax
from jax.experimental import pallas as pl
from jax.experimental.pallas import tpu as pltpu
from jax.experimental.pallas import tpu_sc as plsc

N = 10000
NC = 2
NS = 16
NW = NC * NS
CHUNK = 128
ACC_ROWS = 10240
ROWS_PT = ACC_ROWS // NS
ZCH = ROWS_PT // CHUNK

_f32 = jnp.float32


def _mesh():
    return plsc.VectorSubcoreMesh(core_axis_name="c", subcore_axis_name="s")


def _make_spmm(F, CPT):
    @functools.partial(
        pl.kernel,
        out_type=jax.ShapeDtypeStruct((NC, ACC_ROWS, F), _f32),
        mesh=_mesh(),
        compiler_params=pltpu.CompilerParams(use_tc_tiling_on_sc=(F % 128 == 0)),
        scratch_types=[
            pltpu.VMEM((CPT, CHUNK), jnp.int32),
            pltpu.VMEM((2, CHUNK), jnp.int32),
            pltpu.VMEM((2, CHUNK), jnp.int32),
            pltpu.VMEM((CHUNK, F), _f32),
            pltpu.VMEM((CHUNK, F), _f32),
            pltpu.VMEM_SHARED((ACC_ROWS, F), _f32),
            pltpu.SemaphoreType.DMA,
            pltpu.SemaphoreType.DMA,
            pltpu.SemaphoreType.DMA,
            pltpu.SemaphoreType.DMA,
        ],
    )
    def spmm(support_hbm, packed_hbm, out_hbm,
             packed_v, rc0, rc1, gbuf0, gbuf1, acc, sem0, sem1, ssem0, ssem1):
        c = lax.axis_index("c")
        s = lax.axis_index("s")
        wid = c * NS + s
        pltpu.sync_copy(packed_hbm.at[wid], packed_v)

        zv = jnp.zeros((16,), _f32)

        def zrow(i, carry):
            for j in range(F // 16):
                gbuf0[i, pl.ds(j * 16, 16)] = zv
            return carry

        lax.fori_loop(0, CHUNK, zrow, 0)
        for k in range(ZCH):
            pltpu.sync_copy(gbuf0, acc.at[pl.ds(s * ROWS_PT + k * CHUNK, CHUNK)])
        plsc.subcore_barrier()

        def prep(j, rc, gbuf, sem):
            for k in range(CHUNK // 16):
                pv = packed_v[j, pl.ds(k * 16, 16)]
                rc[0, pl.ds(k * 16, 16)] = lax.shift_right_logical(pv, 14)
                rc[1, pl.ds(k * 16, 16)] = lax.bitwise_and(pv, 16383)
            pltpu.async_copy(support_hbm.at[rc.at[1]], gbuf, sem)

        def _drain_g(sem, buf):
            pltpu.make_async_copy(support_hbm.at[pl.ds(0, CHUNK)], buf, sem).wait()

        def _drain_s(sem, rc, buf):
            pltpu.make_async_copy(buf, acc.at[rc.at[0]], sem).wait()

        prep(0, rc0, gbuf0, sem0)
        prep(1, rc1, gbuf1, sem1)

        def pair(i, carry):
            j = 2 * i
            _drain_g(sem0, gbuf0)
            pltpu.async_copy(gbuf0, acc.at[rc0.at[0]], ssem0, add=True)
            _drain_g(sem1, gbuf1)
            pltpu.async_copy(gbuf1, acc.at[rc1.at[0]], ssem1, add=True)
            _drain_s(ssem0, rc0, gbuf0)
            nxt0 = jnp.minimum(j + 2, CPT - 1)
            prep(nxt0, rc0, gbuf0, sem0)
            _drain_s(ssem1, rc1, gbuf1)
            nxt1 = jnp.minimum(j + 3, CPT - 1)
            prep(nxt1, rc1, gbuf1, sem1)
            return carry

        lax.fori_loop(0, CPT // 2, pair, 0)
        _drain_g(sem0, gbuf0)
        _drain_g(sem1, gbuf1)
        plsc.subcore_barrier()
        pltpu.sync_copy(acc.at[pl.ds(s * ROWS_PT, ROWS_PT)],
                        out_hbm.at[c, pl.ds(s * ROWS_PT, ROWS_PT)])

    return spmm


def _make_hist(CPT):
    @functools.partial(
        pl.kernel,
        out_type=jax.ShapeDtypeStruct((NC, ACC_ROWS), _f32),
        mesh=_mesh(),
        scratch_types=[
            pltpu.VMEM((CPT, CHUNK), jnp.int32),
            pltpu.VMEM((CHUNK,), jnp.int32),
            pltpu.VMEM((CHUNK,), _f32),
            pltpu.VMEM((ROWS_PT,), _f32),
            pltpu.VMEM_SHARED((ACC_ROWS,), _f32),
        ],
    )
    def hist(packed_hbm, out_hbm, packed_v, ridx, ones_v, zb, acc):
        c = lax.axis_index("c")
        s = lax.axis_index("s")
        wid = c * NS + s
        pltpu.sync_copy(packed_hbm.at[wid], packed_v)
        ov = jnp.ones((16,), _f32)
        zv = jnp.zeros((16,), _f32)
        for j in range(CHUNK // 16):
            ones_v[pl.ds(j * 16, 16)] = ov
        for j in range(ROWS_PT // 16):
            zb[pl.ds(j * 16, 16)] = zv
        pltpu.sync_copy(zb, acc.at[pl.ds(s * ROWS_PT, ROWS_PT)])
        plsc.subcore_barrier()

        def step(j, carry):
            for k in range(CHUNK // 16):
                pv = packed_v[j, pl.ds(k * 16, 16)]
                ridx[pl.ds(k * 16, 16)] = lax.shift_right_logical(pv, 14)
            pltpu.sync_copy(ones_v, acc.at[ridx], add=True)
            return carry

        lax.fori_loop(0, CPT, step, 0)
        plsc.subcore_barrier()
        pltpu.sync_copy(acc.at[pl.ds(s * ROWS_PT, ROWS_PT)],
                        out_hbm.at[c, pl.ds(s * ROWS_PT, ROWS_PT)])

    return hist


_CONTRACT_LAST = (((1,), (1,)), ((), ()))


def _mm1_body(x_ref, w_ref, o_ref):
    o_ref[...] = lax.dot_general(x_ref[...], w_ref[...], _CONTRACT_LAST,
                                 preferred_element_type=_f32)


def _dense1(x, W1):
    return pl.pallas_call(
        _mm1_body,
        grid=(10,),
        in_specs=[pl.BlockSpec((1000, 128), lambda i: (i, 0)),
                  pl.BlockSpec((128, 128), lambda i: (0, 0))],
        out_specs=pl.BlockSpec((1000, 128), lambda i: (i, 0)),
        out_shape=jax.ShapeDtypeStruct((N, 128), _f32),
    )(x, W1)


def _mid_body(p_ref, s1_ref, d0_ref, d1_ref, w2_ref, o_ref):
    inv = 1.0 / (1.0 + d0_ref[...] + d1_ref[...])
    h = jnp.maximum((p_ref[0] + p_ref[1] + s1_ref[...]) * inv, 0.0)
    o_ref[...] = lax.dot_general(h, w2_ref[...], _CONTRACT_LAST,
                                 preferred_element_type=_f32)


def _dense_mid(p, s1, d0, d1, W2):
    return pl.pallas_call(
        _mid_body,
        grid=(10,),
        in_specs=[pl.BlockSpec((2, 1000, 128), lambda i: (0, i, 0)),
                  pl.BlockSpec((1000, 128), lambda i: (i, 0)),
                  pl.BlockSpec((1000, 1), lambda i: (i, 0)),
                  pl.BlockSpec((1000, 1), lambda i: (i, 0)),
                  pl.BlockSpec((64, 128), lambda i: (0, 0))],
        out_specs=pl.BlockSpec((1000, 64), lambda i: (i, 0)),
        out_shape=jax.ShapeDtypeStruct((N, 64), _f32),
    )(p, s1, d0, d1, W2)


def _fin_body(q_ref, s2_ref, d0_ref, d1_ref, o_ref):
    inv = 1.0 / (1.0 + d0_ref[...] + d1_ref[...])
    o_ref[...] = (q_ref[0] + q_ref[1] + s2_ref[...]) * inv


def _dense_fin(q, s2, d0, d1):
    return pl.pallas_call(
        _fin_body,
        grid=(10,),
        in_specs=[pl.BlockSpec((2, 1000, 64), lambda i: (0, i, 0)),
                  pl.BlockSpec((1000, 64), lambda i: (i, 0)),
                  pl.BlockSpec((1000, 1), lambda i: (i, 0)),
                  pl.BlockSpec((1000, 1), lambda i: (i, 0))],
        out_specs=pl.BlockSpec((1000, 64), lambda i: (i, 0)),
        out_shape=jax.ShapeDtypeStruct((N, 64), _f32),
    )(q, s2, d0, d1)


def kernel(x, edge_index, W1, W2):
    E = edge_index.shape[1]
    per_chunk_round = NW * CHUNK
    CPT = -(-E // per_chunk_round)
    CPT += CPT % 2
    EPAD = CPT * per_chunk_round
    pad = EPAD - E
    pidx = jnp.arange(pad, dtype=jnp.int32)
    packed = jnp.concatenate([
        (edge_index[0] << 14) | edge_index[1],
        ((N + pidx % (ACC_ROWS - N)) << 14) | (pidx % N),
    ])
    packed_r = packed.reshape(NW, CPT, CHUNK)

    deg = _make_hist(CPT)(packed_r)
    d0 = deg[0, :N, None]
    d1 = deg[1, :N, None]

    s1 = _dense1(x, W1)
    p = _make_spmm(128, CPT)(s1, packed_r)
    s2 = _dense_mid(p, s1, d0, d1, W2)
    q = _make_spmm(64, CPT)(s2, packed_r)
    return _dense_fin(q, s2, d0, d1)


# speedup vs baseline: 1.2197x; 1.2197x over previous
import functools

import jax
import jax.numpy as jnp
from jax import lax
from jax.experimental import pallas as pl
from jax.experimental.pallas import tpu as pltpu
from jax.experimental.pallas import tpu_sc as plsc

N = 10000
NC = 2
NS = 16
NW = NC * NS
CHUNK = 128
ACC_ROWS = 10240
ROWS_PT = ACC_ROWS // NS
ZCH = ROWS_PT // CHUNK

_f32 = jnp.float32


def _mesh():
    return plsc.VectorSubcoreMesh(core_axis_name="c", subcore_axis_name="s")


def _make_spmm(F, CPT):
    @functools.partial(
        pl.kernel,
        out_type=jax.ShapeDtypeStruct((NC, ACC_ROWS, F), _f32),
        mesh=_mesh(),
        compiler_params=pltpu.CompilerParams(use_tc_tiling_on_sc=(F % 128 == 0)),
        scratch_types=[
            pltpu.VMEM((CPT, CHUNK), jnp.int32),
            pltpu.VMEM((2, CHUNK), jnp.int32),
            pltpu.VMEM((2, CHUNK), jnp.int32),
            pltpu.VMEM((CHUNK, F), _f32),
            pltpu.VMEM((CHUNK, F), _f32),
            pltpu.VMEM_SHARED((ACC_ROWS, F), _f32),
            pltpu.SemaphoreType.DMA,
            pltpu.SemaphoreType.DMA,
        ],
    )
    def spmm(support_hbm, packed_hbm, out_hbm,
             packed_v, rc0, rc1, gbuf0, gbuf1, acc, sem0, sem1):
        c = lax.axis_index("c")
        s = lax.axis_index("s")
        wid = c * NS + s
        pltpu.sync_copy(packed_hbm.at[wid], packed_v)

        zv = jnp.zeros((16,), _f32)

        def zrow(i, carry):
            for j in range(F // 16):
                gbuf0[i, pl.ds(j * 16, 16)] = zv
            return carry

        lax.fori_loop(0, CHUNK, zrow, 0)
        for k in range(ZCH):
            pltpu.sync_copy(gbuf0, acc.at[pl.ds(s * ROWS_PT + k * CHUNK, CHUNK)])
        plsc.subcore_barrier()

        def prep(j, rc, gbuf, sem):
            for k in range(CHUNK // 16):
                pv = packed_v[j, pl.ds(k * 16, 16)]
                rc[0, pl.ds(k * 16, 16)] = lax.shift_right_logical(pv, 14)
                rc[1, pl.ds(k * 16, 16)] = lax.bitwise_and(pv, 16383)
            pltpu.async_copy(support_hbm.at[rc.at[1]], gbuf, sem)

        def _drain_g(sem, buf):
            pltpu.make_async_copy(support_hbm.at[pl.ds(0, CHUNK)], buf, sem).wait()

        prep(0, rc0, gbuf0, sem0)

        def pair(i, carry):
            j = 2 * i
            prep(j + 1, rc1, gbuf1, sem1)
            _drain_g(sem0, gbuf0)
            pltpu.sync_copy(gbuf0, acc.at[rc0.at[0]], add=True)
            nxt = jnp.minimum(j + 2, CPT - 1)
            prep(nxt, rc0, gbuf0, sem0)
            _drain_g(sem1, gbuf1)
            pltpu.sync_copy(gbuf1, acc.at[rc1.at[0]], add=True)
            return carry

        lax.fori_loop(0, CPT // 2, pair, 0)
        _drain_g(sem0, gbuf0)
        plsc.subcore_barrier()
        pltpu.sync_copy(acc.at[pl.ds(s * ROWS_PT, ROWS_PT)],
                        out_hbm.at[c, pl.ds(s * ROWS_PT, ROWS_PT)])

    return spmm


def _make_hist(CPT):
    @functools.partial(
        pl.kernel,
        out_type=jax.ShapeDtypeStruct((NC, ACC_ROWS), _f32),
        mesh=_mesh(),
        scratch_types=[
            pltpu.VMEM((CPT, CHUNK), jnp.int32),
            pltpu.VMEM((CHUNK,), jnp.int32),
            pltpu.VMEM((CHUNK,), _f32),
            pltpu.VMEM((ROWS_PT,), _f32),
            pltpu.VMEM_SHARED((ACC_ROWS,), _f32),
        ],
    )
    def hist(packed_hbm, out_hbm, packed_v, ridx, ones_v, zb, acc):
        c = lax.axis_index("c")
        s = lax.axis_index("s")
        wid = c * NS + s
        pltpu.sync_copy(packed_hbm.at[wid], packed_v)
        ov = jnp.ones((16,), _f32)
        zv = jnp.zeros((16,), _f32)
        for j in range(CHUNK // 16):
            ones_v[pl.ds(j * 16, 16)] = ov
        for j in range(ROWS_PT // 16):
            zb[pl.ds(j * 16, 16)] = zv
        pltpu.sync_copy(zb, acc.at[pl.ds(s * ROWS_PT, ROWS_PT)])
        plsc.subcore_barrier()

        def step(j, carry):
            for k in range(CHUNK // 16):
                pv = packed_v[j, pl.ds(k * 16, 16)]
                ridx[pl.ds(k * 16, 16)] = lax.shift_right_logical(pv, 14)
            pltpu.sync_copy(ones_v, acc.at[ridx], add=True)
            return carry

        lax.fori_loop(0, CPT, step, 0)
        plsc.subcore_barrier()
        pltpu.sync_copy(acc.at[pl.ds(s * ROWS_PT, ROWS_PT)],
                        out_hbm.at[c, pl.ds(s * ROWS_PT, ROWS_PT)])

    return hist


_CONTRACT_LAST = (((1,), (1,)), ((), ()))


def _mm1_body(x_ref, w_ref, o_ref):
    o_ref[...] = lax.dot_general(x_ref[...], w_ref[...], _CONTRACT_LAST,
                                 preferred_element_type=_f32)


def _dense1(x, W1):
    return pl.pallas_call(
        _mm1_body,
        grid=(10,),
        in_specs=[pl.BlockSpec((1000, 128), lambda i: (i, 0)),
                  pl.BlockSpec((128, 128), lambda i: (0, 0))],
        out_specs=pl.BlockSpec((1000, 128), lambda i: (i, 0)),
        out_shape=jax.ShapeDtypeStruct((N, 128), _f32),
    )(x, W1)


def _mid_body(p_ref, s1_ref, d0_ref, d1_ref, w2_ref, o_ref):
    inv = 1.0 / (1.0 + d0_ref[...] + d1_ref[...])
    h = jnp.maximum((p_ref[0] + p_ref[1] + s1_ref[...]) * inv, 0.0)
    o_ref[...] = lax.dot_general(h, w2_ref[...], _CONTRACT_LAST,
                                 preferred_element_type=_f32)


def _dense_mid(p, s1, d0, d1, W2):
    return pl.pallas_call(
        _mid_body,
        grid=(10,),
        in_specs=[pl.BlockSpec((2, 1000, 128), lambda i: (0, i, 0)),
                  pl.BlockSpec((1000, 128), lambda i: (i, 0)),
                  pl.BlockSpec((1000, 1), lambda i: (i, 0)),
                  pl.BlockSpec((1000, 1), lambda i: (i, 0)),
                  pl.BlockSpec((64, 128), lambda i: (0, 0))],
        out_specs=pl.BlockSpec((1000, 64), lambda i: (i, 0)),
        out_shape=jax.ShapeDtypeStruct((N, 64), _f32),
    )(p, s1, d0, d1, W2)


def _fin_body(q_ref, s2_ref, d0_ref, d1_ref, o_ref):
    inv = 1.0 / (1.0 + d0_ref[...] + d1_ref[...])
    o_ref[...] = (q_ref[0] + q_ref[1] + s2_ref[...]) * inv


def _dense_fin(q, s2, d0, d1):
    return pl.pallas_call(
        _fin_body,
        grid=(10,),
        in_specs=[pl.BlockSpec((2, 1000, 64), lambda i: (0, i, 0)),
                  pl.BlockSpec((1000, 64), lambda i: (i, 0)),
                  pl.BlockSpec((1000, 1), lambda i: (i, 0)),
                  pl.BlockSpec((1000, 1), lambda i: (i, 0))],
        out_specs=pl.BlockSpec((1000, 64), lambda i: (i, 0)),
        out_shape=jax.ShapeDtypeStruct((N, 64), _f32),
    )(q, s2, d0, d1)


def kernel(x, edge_index, W1, W2):
    E = edge_index.shape[1]
    per_chunk_round = NW * CHUNK
    CPT = -(-E // per_chunk_round)
    CPT += CPT % 2
    EPAD = CPT * per_chunk_round
    pad = EPAD - E
    pidx = jnp.arange(pad, dtype=jnp.int32)
    packed = jnp.concatenate([
        (edge_index[0] << 14) | edge_index[1],
        ((N + pidx % (ACC_ROWS - N)) << 14) | (pidx % N),
    ])
    packed_r = packed.reshape(NW, CPT, CHUNK)

    deg = _make_hist(CPT)(packed_r)
    d0 = deg[0, :N, None]
    d1 = deg[1, :N, None]

    s1 = _dense1(x, W1)
    p = _make_spmm(128, CPT)(s1, packed_r)
    s2 = _dense_mid(p, s1, d0, d1, W2)
    q = _make_spmm(64, CPT)(s2, packed_r)
    return _dense_fin(q, s2, d0, d1)


# speedup vs baseline: 1.2967x; 1.0632x over previous
import functools

import jax
import jax.numpy as jnp
from jax import lax
from jax.experimental import pallas as pl
from jax.experimental.pallas import tpu as pltpu
from jax.experimental.pallas import tpu_sc as plsc

N = 10000
NC = 2
NS = 16
NW = NC * NS
CHUNK = 128
ACC_ROWS = 10240
ROWS_PT = ACC_ROWS // NS
ZCH = ROWS_PT // CHUNK

_f32 = jnp.float32


def _mesh():
    return plsc.VectorSubcoreMesh(core_axis_name="c", subcore_axis_name="s")


def _make_spmm(F, CPT, NBUF):
    assert CPT % NBUF == 0

    @functools.partial(
        pl.kernel,
        out_type=jax.ShapeDtypeStruct((NC, ACC_ROWS, F), _f32),
        mesh=_mesh(),
        compiler_params=pltpu.CompilerParams(use_tc_tiling_on_sc=(F % 128 == 0)),
        scratch_types=(
            [pltpu.VMEM((CPT, CHUNK), jnp.int32)]
            + [pltpu.VMEM((2, CHUNK), jnp.int32) for _ in range(NBUF)]
            + [pltpu.VMEM((CHUNK, F), _f32) for _ in range(NBUF)]
            + [pltpu.VMEM_SHARED((ACC_ROWS, F), _f32)]
            + [pltpu.SemaphoreType.DMA for _ in range(NBUF)]
        ),
    )
    def spmm(support_hbm, packed_hbm, out_hbm, packed_v, *rest):
        rcs = rest[:NBUF]
        gbufs = rest[NBUF:2 * NBUF]
        acc = rest[2 * NBUF]
        sems = rest[2 * NBUF + 1:]
        c = lax.axis_index("c")
        s = lax.axis_index("s")
        wid = c * NS + s
        pltpu.sync_copy(packed_hbm.at[wid], packed_v)

        zv = jnp.zeros((16,), _f32)

        def zrow(i, carry):
            for j in range(F // 16):
                gbufs[0][i, pl.ds(j * 16, 16)] = zv
            return carry

        lax.fori_loop(0, CHUNK, zrow, 0)
        for k in range(ZCH):
            pltpu.sync_copy(gbufs[0], acc.at[pl.ds(s * ROWS_PT + k * CHUNK, CHUNK)])
        plsc.subcore_barrier()

        def prep(j, b):
            for k in range(CHUNK // 16):
                pv = packed_v[j, pl.ds(k * 16, 16)]
                rcs[b][0, pl.ds(k * 16, 16)] = lax.shift_right_logical(pv, 14)
                rcs[b][1, pl.ds(k * 16, 16)] = lax.bitwise_and(pv, 16383)
            pltpu.async_copy(support_hbm.at[rcs[b].at[1]], gbufs[b], sems[b])

        def _drain_g(b):
            pltpu.make_async_copy(support_hbm.at[pl.ds(0, CHUNK)],
                                  gbufs[b], sems[b]).wait()

        for b in range(NBUF - 1):
            prep(b, b)

        def group(i, carry):
            j0 = i * NBUF
            for b in range(NBUF):
                j = j0 + b
                prep(jnp.minimum(j + NBUF - 1, CPT - 1), (b - 1) % NBUF)
                _drain_g(b)
                pltpu.sync_copy(gbufs[b], acc.at[rcs[b].at[0]], add=True)
            return carry

        lax.fori_loop(0, CPT // NBUF, group, 0)
        for b in range(NBUF - 1):
            _drain_g(b)
        plsc.subcore_barrier()
        pltpu.sync_copy(acc.at[pl.ds(s * ROWS_PT, ROWS_PT)],
                        out_hbm.at[c, pl.ds(s * ROWS_PT, ROWS_PT)])

    return spmm


def _make_hist(CPT):
    @functools.partial(
        pl.kernel,
        out_type=jax.ShapeDtypeStruct((NC, ACC_ROWS), _f32),
        mesh=_mesh(),
        scratch_types=[
            pltpu.VMEM((CPT, CHUNK), jnp.int32),
            pltpu.VMEM((CHUNK,), jnp.int32),
            pltpu.VMEM((CHUNK,), _f32),
            pltpu.VMEM((ROWS_PT,), _f32),
            pltpu.VMEM_SHARED((ACC_ROWS,), _f32),
        ],
    )
    def hist(packed_hbm, out_hbm, packed_v, ridx, ones_v, zb, acc):
        c = lax.axis_index("c")
        s = lax.axis_index("s")
        wid = c * NS + s
        pltpu.sync_copy(packed_hbm.at[wid], packed_v)
        ov = jnp.ones((16,), _f32)
        zv = jnp.zeros((16,), _f32)
        for j in range(CHUNK // 16):
            ones_v[pl.ds(j * 16, 16)] = ov
        for j in range(ROWS_PT // 16):
            zb[pl.ds(j * 16, 16)] = zv
        pltpu.sync_copy(zb, acc.at[pl.ds(s * ROWS_PT, ROWS_PT)])
        plsc.subcore_barrier()

        def step(j, carry):
            for k in range(CHUNK // 16):
                pv = packed_v[j, pl.ds(k * 16, 16)]
                ridx[pl.ds(k * 16, 16)] = lax.shift_right_logical(pv, 14)
            pltpu.sync_copy(ones_v, acc.at[ridx], add=True)
            return carry

        lax.fori_loop(0, CPT, step, 0)
        plsc.subcore_barrier()
        pltpu.sync_copy(acc.at[pl.ds(s * ROWS_PT, ROWS_PT)],
                        out_hbm.at[c, pl.ds(s * ROWS_PT, ROWS_PT)])

    return hist


_CONTRACT_LAST = (((1,), (1,)), ((), ()))


def _mm1_body(x_ref, w_ref, o_ref):
    o_ref[...] = lax.dot_general(x_ref[...], w_ref[...], _CONTRACT_LAST,
                                 preferred_element_type=_f32)


def _dense1(x, W1):
    return pl.pallas_call(
        _mm1_body,
        grid=(10,),
        in_specs=[pl.BlockSpec((1000, 128), lambda i: (i, 0)),
                  pl.BlockSpec((128, 128), lambda i: (0, 0))],
        out_specs=pl.BlockSpec((1000, 128), lambda i: (i, 0)),
        out_shape=jax.ShapeDtypeStruct((N, 128), _f32),
    )(x, W1)


def _mid_body(p_ref, s1_ref, d0_ref, d1_ref, w2_ref, o_ref):
    inv = 1.0 / (1.0 + d0_ref[...] + d1_ref[...])
    h = jnp.maximum((p_ref[0] + p_ref[1] + s1_ref[...]) * inv, 0.0)
    o_ref[...] = lax.dot_general(h, w2_ref[...], _CONTRACT_LAST,
                                 preferred_element_type=_f32)


def _dense_mid(p, s1, d0, d1, W2):
    return pl.pallas_call(
        _mid_body,
        grid=(10,),
        in_specs=[pl.BlockSpec((2, 1000, 128), lambda i: (0, i, 0)),
                  pl.BlockSpec((1000, 128), lambda i: (i, 0)),
                  pl.BlockSpec((1000, 1), lambda i: (i, 0)),
                  pl.BlockSpec((1000, 1), lambda i: (i, 0)),
                  pl.BlockSpec((64, 128), lambda i: (0, 0))],
        out_specs=pl.BlockSpec((1000, 64), lambda i: (i, 0)),
        out_shape=jax.ShapeDtypeStruct((N, 64), _f32),
    )(p, s1, d0, d1, W2)


def _fin_body(q_ref, s2_ref, d0_ref, d1_ref, o_ref):
    inv = 1.0 / (1.0 + d0_ref[...] + d1_ref[...])
    o_ref[...] = (q_ref[0] + q_ref[1] + s2_ref[...]) * inv


def _dense_fin(q, s2, d0, d1):
    return pl.pallas_call(
        _fin_body,
        grid=(10,),
        in_specs=[pl.BlockSpec((2, 1000, 64), lambda i: (0, i, 0)),
                  pl.BlockSpec((1000, 64), lambda i: (i, 0)),
                  pl.BlockSpec((1000, 1), lambda i: (i, 0)),
                  pl.BlockSpec((1000, 1), lambda i: (i, 0))],
        out_specs=pl.BlockSpec((1000, 64), lambda i: (i, 0)),
        out_shape=jax.ShapeDtypeStruct((N, 64), _f32),
    )(q, s2, d0, d1)


def kernel(x, edge_index, W1, W2):
    E = edge_index.shape[1]
    per_chunk_round = NW * CHUNK
    CPT = -(-E // per_chunk_round)
    CPT = -(-CPT // 4) * 4
    EPAD = CPT * per_chunk_round
    pad = EPAD - E
    pidx = jnp.arange(pad, dtype=jnp.int32)
    packed = jnp.concatenate([
        (edge_index[0] << 14) | edge_index[1],
        ((N + pidx % (ACC_ROWS - N)) << 14) | (pidx % N),
    ])
    packed_r = packed.reshape(NW, CPT, CHUNK)

    deg = _make_hist(CPT)(packed_r)
    d0 = deg[0, :N, None]
    d1 = deg[1, :N, None]

    s1 = _dense1(x, W1)
    p = _make_spmm(128, CPT, 2)(s1, packed_r)
    s2 = _dense_mid(p, s1, d0, d1, W2)
    q = _make_spmm(64, CPT, 4)(s2, packed_r)
    return _dense_fin(q, s2, d0, d1)


# speedup vs baseline: 1.3221x; 1.0196x over previous
import functools

import jax
import jax.numpy as jnp
from jax import lax
from jax.experimental import pallas as pl
from jax.experimental.pallas import tpu as pltpu
from jax.experimental.pallas import tpu_sc as plsc

N = 10000
C_OUT = 64
NC = 2
NS = 16
NW = NC * NS
CHUNK = 128
ACC_ROWS = 10240
ROWS_PT = ACC_ROWS // NS
ZCH = ROWS_PT // CHUNK

_f32 = jnp.float32


def _mesh():
    return plsc.VectorSubcoreMesh(core_axis_name="c", subcore_axis_name="s")


def _make_spmm(F, CPT, NBUF):
    assert CPT % NBUF == 0

    @functools.partial(
        pl.kernel,
        out_type=jax.ShapeDtypeStruct((NC, ACC_ROWS, F), _f32),
        mesh=_mesh(),
        compiler_params=pltpu.CompilerParams(use_tc_tiling_on_sc=(F % 128 == 0)),
        scratch_types=(
            [pltpu.VMEM((CPT, CHUNK), jnp.int32)]
            + [pltpu.VMEM((2, CHUNK), jnp.int32) for _ in range(NBUF)]
            + [pltpu.VMEM((CHUNK, F), _f32) for _ in range(NBUF)]
            + [pltpu.VMEM_SHARED((ACC_ROWS, F), _f32)]
            + [pltpu.SemaphoreType.DMA for _ in range(NBUF)]
        ),
    )
    def spmm(support_hbm, packed_hbm, out_hbm, packed_v, *rest):
        rcs = rest[:NBUF]
        gbufs = rest[NBUF:2 * NBUF]
        acc = rest[2 * NBUF]
        sems = rest[2 * NBUF + 1:]
        c = lax.axis_index("c")
        s = lax.axis_index("s")
        wid = c * NS + s
        pltpu.sync_copy(packed_hbm.at[wid], packed_v)

        zv = jnp.zeros((16,), _f32)

        def zrow(i, carry):
            for j in range(F // 16):
                gbufs[0][i, pl.ds(j * 16, 16)] = zv
            return carry

        lax.fori_loop(0, CHUNK, zrow, 0)
        for k in range(ZCH):
            pltpu.sync_copy(gbufs[0], acc.at[pl.ds(s * ROWS_PT + k * CHUNK, CHUNK)])
        plsc.subcore_barrier()

        def prep(j, b):
            for k in range(CHUNK // 16):
                pv = packed_v[j, pl.ds(k * 16, 16)]
                rcs[b][0, pl.ds(k * 16, 16)] = lax.shift_right_logical(pv, 14)
                rcs[b][1, pl.ds(k * 16, 16)] = lax.bitwise_and(pv, 16383)
            pltpu.async_copy(support_hbm.at[rcs[b].at[1]], gbufs[b], sems[b])

        def _drain_g(b):
            pltpu.make_async_copy(support_hbm.at[pl.ds(0, CHUNK)],
                                  gbufs[b], sems[b]).wait()

        for b in range(NBUF - 1):
            prep(b, b)

        def group(i, carry):
            j0 = i * NBUF
            for b in range(NBUF):
                j = j0 + b
                prep(jnp.minimum(j + NBUF - 1, CPT - 1), (b - 1) % NBUF)
                _drain_g(b)
                pltpu.sync_copy(gbufs[b], acc.at[rcs[b].at[0]], add=True)
            return carry

        lax.fori_loop(0, CPT // NBUF, group, 0)
        for b in range(NBUF - 1):
            _drain_g(b)
        plsc.subcore_barrier()
        pltpu.sync_copy(acc.at[pl.ds(s * ROWS_PT, ROWS_PT)],
                        out_hbm.at[c, pl.ds(s * ROWS_PT, ROWS_PT)])

    return spmm


def _make_hist(CPT):
    @functools.partial(
        pl.kernel,
        out_type=jax.ShapeDtypeStruct((NC, ACC_ROWS), _f32),
        mesh=_mesh(),
        scratch_types=[
            pltpu.VMEM((CPT, CHUNK), jnp.int32),
            pltpu.VMEM((CHUNK,), jnp.int32),
            pltpu.VMEM((CHUNK,), _f32),
            pltpu.VMEM((ROWS_PT,), _f32),
            pltpu.VMEM_SHARED((ACC_ROWS,), _f32),
        ],
    )
    def hist(packed_hbm, out_hbm, packed_v, ridx, ones_v, zb, acc):
        c = lax.axis_index("c")
        s = lax.axis_index("s")
        wid = c * NS + s
        pltpu.sync_copy(packed_hbm.at[wid], packed_v)
        ov = jnp.ones((16,), _f32)
        zv = jnp.zeros((16,), _f32)
        for j in range(CHUNK // 16):
            ones_v[pl.ds(j * 16, 16)] = ov
        for j in range(ROWS_PT // 16):
            zb[pl.ds(j * 16, 16)] = zv
        pltpu.sync_copy(zb, acc.at[pl.ds(s * ROWS_PT, ROWS_PT)])
        plsc.subcore_barrier()

        def step(j, carry):
            for k in range(CHUNK // 16):
                pv = packed_v[j, pl.ds(k * 16, 16)]
                ridx[pl.ds(k * 16, 16)] = lax.shift_right_logical(pv, 14)
            pltpu.sync_copy(ones_v, acc.at[ridx], add=True)
            return carry

        lax.fori_loop(0, CPT, step, 0)
        plsc.subcore_barrier()
        pltpu.sync_copy(acc.at[pl.ds(s * ROWS_PT, ROWS_PT)],
                        out_hbm.at[c, pl.ds(s * ROWS_PT, ROWS_PT)])

    return hist


_CONTRACT_LAST = (((1,), (1,)), ((), ()))


def _mm1_body(x_ref, w_ref, o_ref):
    o_ref[...] = lax.dot_general(x_ref[...], w_ref[...], _CONTRACT_LAST,
                                 preferred_element_type=_f32)


def _dense1(x, W1):
    return pl.pallas_call(
        _mm1_body,
        grid=(10,),
        in_specs=[pl.BlockSpec((1000, 128), lambda i: (i, 0)),
                  pl.BlockSpec((128, 128), lambda i: (0, 0))],
        out_specs=pl.BlockSpec((1000, 128), lambda i: (i, 0)),
        out_shape=jax.ShapeDtypeStruct((N, 128), _f32),
    )(x, W1)


def _mid_body(p_ref, s1_ref, d0_ref, d1_ref, w2_ref, o_ref):
    inv = 1.0 / (1.0 + d0_ref[...] + d1_ref[...])
    h = jnp.maximum((p_ref[0] + p_ref[1] + s1_ref[...]) * inv, 0.0)
    o_ref[...] = lax.dot_general(h, w2_ref[...], _CONTRACT_LAST,
                                 preferred_element_type=_f32)


def _dense_mid(p, s1, d0, d1, W2):
    return pl.pallas_call(
        _mid_body,
        grid=(25,),
        in_specs=[pl.BlockSpec((2, 400, 128), lambda i: (0, i, 0)),
                  pl.BlockSpec((400, 128), lambda i: (i, 0)),
                  pl.BlockSpec((400, 1), lambda i: (i, 0)),
                  pl.BlockSpec((400, 1), lambda i: (i, 0)),
                  pl.BlockSpec((64, 128), lambda i: (0, 0))],
        out_specs=pl.BlockSpec((400, 64), lambda i: (i, 0)),
        out_shape=jax.ShapeDtypeStruct((N, 64), _f32),
    )(p, s1, d0, d1, W2)


def _fin_body(q_ref, s2_ref, invp_ref, o_ref):
    o_ref[...] = (q_ref[0] + q_ref[1] + s2_ref[...]) * invp_ref[...]


def _dense_fin(q_pair, s2_pair, inv_pair):
    return pl.pallas_call(
        _fin_body,
        grid=(5,),
        in_specs=[pl.BlockSpec((2, 1000, 128), lambda i: (0, i, 0)),
                  pl.BlockSpec((1000, 128), lambda i: (i, 0)),
                  pl.BlockSpec((1000, 128), lambda i: (i, 0))],
        out_specs=pl.BlockSpec((1000, 128), lambda i: (i, 0)),
        out_shape=jax.ShapeDtypeStruct((N // 2, 128), _f32),
    )(q_pair, s2_pair, inv_pair)


def kernel(x, edge_index, W1, W2):
    E = edge_index.shape[1]
    per_chunk_round = NW * CHUNK
    CPT = -(-E // per_chunk_round)
    CPT = -(-CPT // 4) * 4
    EPAD = CPT * per_chunk_round
    pad = EPAD - E
    pidx = jnp.arange(pad, dtype=jnp.int32)
    packed = jnp.concatenate([
        (edge_index[0] << 14) | edge_index[1],
        ((N + pidx % (ACC_ROWS - N)) << 14) | (pidx % N),
    ])
    packed_r = packed.reshape(NW, CPT, CHUNK)

    deg = _make_hist(CPT)(packed_r)
    d0 = deg[0, :N, None]
    d1 = deg[1, :N, None]

    inv = 1.0 / (1.0 + deg[0, :N] + deg[1, :N])
    inv_pair = jnp.repeat(inv.reshape(N // 2, 2), C_OUT, axis=1)

    s1 = _dense1(x, W1)
    p = _make_spmm(128, CPT, 2)(s1, packed_r)
    s2 = _dense_mid(p, s1, d0, d1, W2)
    q = _make_spmm(64, CPT, 4)(s2, packed_r)
    q_pair = q.reshape(NC, ACC_ROWS // 2, 128)
    s2_pair = s2.reshape(N // 2, 128)
    out_pair = _dense_fin(q_pair, s2_pair, inv_pair)
    return out_pair.reshape(N, 64)


# speedup vs baseline: 1.3662x; 1.0333x over previous
import functools

import jax
import jax.numpy as jnp
from jax import lax
from jax.experimental import pallas as pl
from jax.experimental.pallas import tpu as pltpu
from jax.experimental.pallas import tpu_sc as plsc

N = 10000
C_OUT = 64
NC = 2
NS = 16
NW = NC * NS
CHUNK = 128
ACC_ROWS = 10240
ROWS_PT = ACC_ROWS // NS
ZCH = ROWS_PT // CHUNK

_f32 = jnp.float32


def _mesh():
    return plsc.VectorSubcoreMesh(core_axis_name="c", subcore_axis_name="s")


def _make_spmm(F, CPT, NBUF):
    assert CPT % NBUF == 0

    @functools.partial(
        pl.kernel,
        out_type=jax.ShapeDtypeStruct((NC, ACC_ROWS, F), _f32),
        mesh=_mesh(),
        compiler_params=pltpu.CompilerParams(use_tc_tiling_on_sc=(F % 128 == 0)),
        scratch_types=(
            [pltpu.VMEM((CPT, CHUNK), jnp.int32)]
            + [pltpu.VMEM((2, CHUNK), jnp.int32) for _ in range(NBUF)]
            + [pltpu.VMEM((CHUNK, F), _f32) for _ in range(NBUF)]
            + [pltpu.VMEM_SHARED((ACC_ROWS, F), _f32)]
            + [pltpu.SemaphoreType.DMA for _ in range(NBUF)]
        ),
    )
    def spmm(support_hbm, packed_hbm, out_hbm, packed_v, *rest):
        rcs = rest[:NBUF]
        gbufs = rest[NBUF:2 * NBUF]
        acc = rest[2 * NBUF]
        sems = rest[2 * NBUF + 1:]
        c = lax.axis_index("c")
        s = lax.axis_index("s")
        wid = c * NS + s
        pltpu.sync_copy(packed_hbm.at[wid], packed_v)

        zv = jnp.zeros((16,), _f32)

        def zrow(i, carry):
            for j in range(F // 16):
                gbufs[0][i, pl.ds(j * 16, 16)] = zv
            return carry

        lax.fori_loop(0, CHUNK, zrow, 0)
        for k in range(ZCH):
            pltpu.sync_copy(gbufs[0], acc.at[pl.ds(s * ROWS_PT + k * CHUNK, CHUNK)])
        plsc.subcore_barrier()

        def prep(j, b):
            for k in range(CHUNK // 16):
                pv = packed_v[j, pl.ds(k * 16, 16)]
                rcs[b][0, pl.ds(k * 16, 16)] = lax.shift_right_logical(pv, 14)
                rcs[b][1, pl.ds(k * 16, 16)] = lax.bitwise_and(pv, 16383)
            pltpu.async_copy(support_hbm.at[rcs[b].at[1]], gbufs[b], sems[b])

        def _drain_g(b):
            pltpu.make_async_copy(support_hbm.at[pl.ds(0, CHUNK)],
                                  gbufs[b], sems[b]).wait()

        for b in range(NBUF - 1):
            prep(b, b)

        def group(i, carry):
            j0 = i * NBUF
            for b in range(NBUF):
                j = j0 + b
                prep(jnp.minimum(j + NBUF - 1, CPT - 1), (b - 1) % NBUF)
                _drain_g(b)
                pltpu.sync_copy(gbufs[b], acc.at[rcs[b].at[0]], add=True)
            return carry

        lax.fori_loop(0, CPT // NBUF, group, 0)
        for b in range(NBUF - 1):
            _drain_g(b)
        plsc.subcore_barrier()
        pltpu.sync_copy(acc.at[pl.ds(s * ROWS_PT, ROWS_PT)],
                        out_hbm.at[c, pl.ds(s * ROWS_PT, ROWS_PT)])

    return spmm


def _make_hist(CPT):
    @functools.partial(
        pl.kernel,
        out_type=jax.ShapeDtypeStruct((NC, ACC_ROWS), _f32),
        mesh=_mesh(),
        scratch_types=[
            pltpu.VMEM((CPT, CHUNK), jnp.int32),
            pltpu.VMEM((CHUNK,), jnp.int32),
            pltpu.VMEM((CHUNK,), _f32),
            pltpu.VMEM((ROWS_PT,), _f32),
            pltpu.VMEM_SHARED((ACC_ROWS,), _f32),
        ],
    )
    def hist(packed_hbm, out_hbm, packed_v, ridx, ones_v, zb, acc):
        c = lax.axis_index("c")
        s = lax.axis_index("s")
        wid = c * NS + s
        pltpu.sync_copy(packed_hbm.at[wid], packed_v)
        ov = jnp.ones((16,), _f32)
        zv = jnp.zeros((16,), _f32)
        for j in range(CHUNK // 16):
            ones_v[pl.ds(j * 16, 16)] = ov
        for j in range(ROWS_PT // 16):
            zb[pl.ds(j * 16, 16)] = zv
        pltpu.sync_copy(zb, acc.at[pl.ds(s * ROWS_PT, ROWS_PT)])
        plsc.subcore_barrier()

        def step(j, carry):
            for k in range(CHUNK // 16):
                pv = packed_v[j, pl.ds(k * 16, 16)]
                ridx[pl.ds(k * 16, 16)] = lax.shift_right_logical(pv, 14)
            pltpu.sync_copy(ones_v, acc.at[ridx], add=True)
            return carry

        lax.fori_loop(0, CPT, step, 0)
        plsc.subcore_barrier()
        pltpu.sync_copy(acc.at[pl.ds(s * ROWS_PT, ROWS_PT)],
                        out_hbm.at[c, pl.ds(s * ROWS_PT, ROWS_PT)])

    return hist


_CONTRACT_LAST = (((1,), (1,)), ((), ()))


def _mm1_body(x_ref, w_ref, o_ref):
    o_ref[...] = lax.dot_general(x_ref[...], w_ref[...], _CONTRACT_LAST,
                                 preferred_element_type=_f32)


def _dense1(x, W1):
    return pl.pallas_call(
        _mm1_body,
        grid=(10,),
        in_specs=[pl.BlockSpec((1000, 128), lambda i: (i, 0)),
                  pl.BlockSpec((128, 128), lambda i: (0, 0))],
        out_specs=pl.BlockSpec((1000, 128), lambda i: (i, 0)),
        out_shape=jax.ShapeDtypeStruct((N, 128), _f32),
    )(x, W1)


def _mid_body(p_ref, s1_ref, d0_ref, d1_ref, w2_ref, o_ref):
    inv = 1.0 / (1.0 + d0_ref[...] + d1_ref[...])
    h = jnp.maximum((p_ref[0] + p_ref[1] + s1_ref[...]) * inv, 0.0)
    o_ref[...] = lax.dot_general(h, w2_ref[...], _CONTRACT_LAST,
                                 preferred_element_type=_f32)


def _dense_mid(p, s1, d0, d1, W2):
    return pl.pallas_call(
        _mid_body,
        grid=(10,),
        in_specs=[pl.BlockSpec((2, 1000, 128), lambda i: (0, i, 0)),
                  pl.BlockSpec((1000, 128), lambda i: (i, 0)),
                  pl.BlockSpec((1000, 1), lambda i: (i, 0)),
                  pl.BlockSpec((1000, 1), lambda i: (i, 0)),
                  pl.BlockSpec((64, 128), lambda i: (0, 0))],
        out_specs=pl.BlockSpec((1000, 64), lambda i: (i, 0)),
        out_shape=jax.ShapeDtypeStruct((N, 64), _f32),
    )(p, s1, d0, d1, W2)


def _fin_body(q_ref, s2_ref, invp_ref, o_ref):
    o_ref[...] = (q_ref[0] + q_ref[1] + s2_ref[...]) * invp_ref[...]


def _dense_fin(q_pair, s2_pair, inv_pair):
    return pl.pallas_call(
        _fin_body,
        grid=(5,),
        in_specs=[pl.BlockSpec((2, 1000, 128), lambda i: (0, i, 0)),
                  pl.BlockSpec((1000, 128), lambda i: (i, 0)),
                  pl.BlockSpec((1000, 128), lambda i: (i, 0))],
        out_specs=pl.BlockSpec((1000, 128), lambda i: (i, 0)),
        out_shape=jax.ShapeDtypeStruct((N // 2, 128), _f32),
    )(q_pair, s2_pair, inv_pair)


def kernel(x, edge_index, W1, W2):
    E = edge_index.shape[1]
    per_chunk_round = NW * CHUNK
    CPT = -(-E // per_chunk_round)
    CPT = -(-CPT // 4) * 4
    EPAD = CPT * per_chunk_round
    pad = EPAD - E
    pidx = jnp.arange(pad, dtype=jnp.int32)
    packed = jnp.concatenate([
        (edge_index[0] << 14) | edge_index[1],
        ((N + pidx % (ACC_ROWS - N)) << 14) | (pidx % N),
    ])
    packed_r = packed.reshape(NW, CPT, CHUNK)

    deg = _make_hist(CPT)(packed_r)
    d0 = deg[0, :N, None]
    d1 = deg[1, :N, None]

    inv = 1.0 / (1.0 + deg[0, :N] + deg[1, :N])
    inv_pair = jnp.repeat(inv.reshape(N // 2, 2), C_OUT, axis=1)

    s1 = _dense1(x, W1)
    p = _make_spmm(128, CPT, 2)(s1, packed_r)
    s2 = _dense_mid(p, s1, d0, d1, W2)
    q = _make_spmm(64, CPT, 4)(s2, packed_r)
    q_pair = q.reshape(NC, ACC_ROWS // 2, 128)
    s2_pair = s2.reshape(N // 2, 128)
    out_pair = _dense_fin(q_pair, s2_pair, inv_pair)
    return out_pair.reshape(N, 64)
